# baseline (speedup 1.0000x reference)
import functools

import jax
import jax.numpy as jnp
from jax import lax
from jax.experimental import pallas as pl
from jax.experimental.pallas import tpu as pltpu
from jax.experimental.pallas import tpu_sc as plsc

B, L, D = 16, 2048, 128
NC, NS, LANES = 2, 16, 16
NW = NC * NS
CH = 128
NCH = (B * L) // (NW * CH)
NSLOT = 4

_mesh = plsc.VectorSubcoreMesh(core_axis_name="c", subcore_axis_name="s")


@functools.partial(
    pl.kernel,
    out_type=jax.ShapeDtypeStruct((B * L, D), jnp.float32),
    mesh=_mesh,
    scratch_types=[
    ] + [pltpu.VMEM((L + LANES,), jnp.int32)] * NCH + [
        pltpu.VMEM((NCH, CH), jnp.int32),
        pltpu.VMEM((NSLOT, CH, D), jnp.float32),
        pltpu.SemaphoreType.DMA,
        pltpu.SemaphoreType.DMA,
        pltpu.SemaphoreType.DMA,
        pltpu.SemaphoreType.DMA,
        pltpu.SemaphoreType.DMA,
        pltpu.SemaphoreType.DMA,
        pltpu.SemaphoreType.DMA,
        pltpu.SemaphoreType.DMA,
        pltpu.SemaphoreType.DMA,
    ],
)
def _flatten_sc(mask_hbm, ex_hbm, out_ex,
                mv0, mv1, mv2, mv3, mv4, mv5, mv6, mv7,
                idx_v, rows_v, msem,
                gsem0, gsem1, gsem2, gsem3,
                wsem0, wsem1, wsem2, wsem3):
    mask_v = (mv0, mv1, mv2, mv3, mv4, mv5, mv6, mv7)
    gsem = (gsem0, gsem1, gsem2, gsem3)
    wsem = (wsem0, wsem1, wsem2, wsem3)
    wid = lax.axis_index("s") * NC + lax.axis_index("c")

    bgs = [(wid + k) % 16 for k in range(NCH)]
    cls = [((wid // 16) * 8 + (wid % 16) * 8 + 5 * k) % 16 for k in range(NCH)]

    for k in range(NCH):
        pltpu.async_copy(mask_hbm.at[bgs[k]],
                         mask_v[k].at[pl.ds(0, L)], msem)
    for k in range(NCH):
        pltpu.make_async_copy(mask_hbm.at[0],
                              mask_v[k].at[pl.ds(0, L)], msem).wait()

    def _num_valid(k):
        def _bs_body(_, carry):
            lo, hi = carry
            mid = (lo + hi) // 2
            go_right = mask_v[k][pl.ds(mid, LANES)][0] != 0
            return (jnp.where(go_right, mid + 1, lo),
                    jnp.where(go_right, hi, mid))

        nv, _ = lax.fori_loop(0, 11, _bs_body,
                              (jnp.int32(0), jnp.int32(L)))
        return nv

    nvs = [_num_valid(k) for k in range(NCH)]

    lane = lax.iota(jnp.int32, 16)

    def _fire(k, s):
        jb = cls[k] * CH
        base = bgs[k] * L
        is_identity = (jb + CH) <= nvs[k]

        @pl.when(is_identity)
        def _():
            pltpu.async_copy(
                ex_hbm.at[pl.ds(base + jb, CH)], rows_v.at[s], gsem[s])

        @pl.when(jnp.logical_not(is_identity))
        def _():
            for i in range(CH // LANES):
                j = jb + i * LANES + lane
                idx_v[k, pl.ds(i * LANES, LANES)] = base + lax.rem(j, nvs[k])
            pltpu.async_copy(ex_hbm.at[idx_v.at[k]], rows_v.at[s], gsem[s])

    def _wait_gather(s):
        pltpu.make_async_copy(
            ex_hbm.at[pl.ds(0, CH)], rows_v.at[s], gsem[s]).wait()

    writes = [None] * NCH
    for k in range(NSLOT):
        _fire(k, k % NSLOT)
    for k in range(NCH):
        s = k % NSLOT
        _wait_gather(s)
        out0 = bgs[k] * L + cls[k] * CH
        writes[k] = pltpu.async_copy(
            rows_v.at[s], out_ex.at[pl.ds(out0, CH)], wsem[s])
        if k + NSLOT < NCH:
            writes[k].wait()
            _fire(k + NSLOT, s)
    for k in range(NCH - NSLOT, NCH):
        writes[k].wait()


def _ctx_body(ctx_ref, out_ref):
    row = ctx_ref[pl.ds(pl.program_id(0), 1), :]
    out_ref[...] = jnp.broadcast_to(row, out_ref.shape)


_ctx_broadcast = pl.pallas_call(
    _ctx_body,
    grid=(B,),
    in_specs=[pl.BlockSpec((B, D), lambda i: (0, 0))],
    out_specs=pl.BlockSpec((L, D), lambda i: (i, 0)),
    out_shape=jax.ShapeDtypeStruct((B * L, D), jnp.float32),
)


def kernel(context_feature, example_feature, list_mask):
    mask_i32 = list_mask.astype(jnp.int32)
    ex2d = example_feature.reshape(B * L, D)
    flat_ex = _flatten_sc(mask_i32, ex2d)
    flat_ctx = _ctx_broadcast(context_feature)
    return flat_ctx, flat_ex


# speedup vs baseline: 1.1504x; 1.1504x over previous
import functools

import jax
import jax.numpy as jnp
from jax import lax
from jax.experimental import pallas as pl
from jax.experimental.pallas import tpu as pltpu
from jax.experimental.pallas import tpu_sc as plsc

B, L, D = 16, 2048, 128
NC, NS, LANES = 2, 16, 16
NW = NC * NS
RW = (B * L) // NW
CH = 128
NCH = RW // CH
NSLOT = 4
WPB = L // 4
ALL_VALID = 0x01010101

_mesh = plsc.VectorSubcoreMesh(core_axis_name="c", subcore_axis_name="s")


@functools.partial(
    pl.kernel,
    out_type=jax.ShapeDtypeStruct((B * L, D), jnp.float32),
    mesh=_mesh,
    scratch_types=[
        pltpu.VMEM((WPB + LANES,), jnp.int32),
        pltpu.VMEM((NCH, CH), jnp.int32),
        pltpu.VMEM((NSLOT, CH, D), jnp.float32),
        pltpu.SemaphoreType.DMA,
        pltpu.SemaphoreType.DMA,
        pltpu.SemaphoreType.DMA,
        pltpu.SemaphoreType.DMA,
        pltpu.SemaphoreType.DMA,
        pltpu.SemaphoreType.DMA,
        pltpu.SemaphoreType.DMA,
        pltpu.SemaphoreType.DMA,
    ],
)
def _flatten_sc(maskw_hbm, ex_hbm, out_ex,
                maskw_v, idx_v, rows_v,
                gsem0, gsem1, gsem2, gsem3,
                wsem0, wsem1, wsem2, wsem3):
    gsem = (gsem0, gsem1, gsem2, gsem3)
    wsem = (wsem0, wsem1, wsem2, wsem3)
    wid = lax.axis_index("s") * NC + lax.axis_index("c")
    b = wid // 2
    half = wid % 2
    row0 = b * L + half * RW
    jbase = half * RW

    pltpu.sync_copy(maskw_hbm.at[b], maskw_v.at[pl.ds(0, WPB)])

    def _bs_body(_, carry):
        lo, hi = carry
        mid = (lo + hi) // 2
        go_right = maskw_v[pl.ds(mid, LANES)][0] == ALL_VALID
        return (jnp.where(go_right, mid + 1, lo),
                jnp.where(go_right, hi, mid))

    lo, _ = lax.fori_loop(0, 9, _bs_body, (jnp.int32(0), jnp.int32(WPB)))
    wsel = jnp.minimum(lo, WPB - 1)
    w = maskw_v[pl.ds(wsel, LANES)][0]
    extra = (w & 1) + ((w >> 8) & 1) + ((w >> 16) & 1) + ((w >> 24) & 1)
    nv = 4 * wsel + extra

    lane = lax.iota(jnp.int32, 16)

    def _build(c):
        for i in range(CH // LANES):
            j = jbase + c * CH + i * LANES + lane
            idx_v[c, pl.ds(i * LANES, LANES)] = b * L + lax.rem(j, nv)

    def _fire(c, s):
        is_identity = (jbase + c * CH + CH) <= nv

        @pl.when(is_identity)
        def _():
            pltpu.async_copy(
                ex_hbm.at[pl.ds(row0 + c * CH, CH)], rows_v.at[s], gsem[s])

        @pl.when(jnp.logical_not(is_identity))
        def _():
            pltpu.async_copy(ex_hbm.at[idx_v.at[c]], rows_v.at[s], gsem[s])

    def _wait_gather(s):
        pltpu.make_async_copy(
            ex_hbm.at[pl.ds(0, CH)], rows_v.at[s], gsem[s]).wait()

    writes = [None] * NCH
    for c in range(NSLOT):
        _build(c)
        _fire(c, c % NSLOT)
    for c in range(NSLOT, NCH):
        _build(c)
    for c in range(NCH):
        s = c % NSLOT
        _wait_gather(s)
        writes[c] = pltpu.async_copy(
            rows_v.at[s], out_ex.at[pl.ds(row0 + c * CH, CH)], wsem[s])
        if c + NSLOT < NCH:
            writes[c].wait()
            _fire(c + NSLOT, s)
    for c in range(NCH - NSLOT, NCH):
        writes[c].wait()


def _ctx_body(ctx_ref, out_ref):
    row = ctx_ref[pl.ds(pl.program_id(0), 1), :]
    out_ref[...] = jnp.broadcast_to(row, out_ref.shape)


_ctx_broadcast = pl.pallas_call(
    _ctx_body,
    grid=(B,),
    in_specs=[pl.BlockSpec((B, D), lambda i: (0, 0))],
    out_specs=pl.BlockSpec((L, D), lambda i: (i, 0)),
    out_shape=jax.ShapeDtypeStruct((B * L, D), jnp.float32),
)


def kernel(context_feature, example_feature, list_mask):
    maskw = list_mask.view(jnp.int32)
    ex2d = example_feature.reshape(B * L, D)
    flat_ex = _flatten_sc(maskw, ex2d)
    flat_ctx = _ctx_broadcast(context_feature)
    return flat_ctx, flat_ex


# speedup vs baseline: 1.1737x; 1.0202x over previous
import functools

import jax
import jax.numpy as jnp
from jax import lax
from jax.experimental import pallas as pl
from jax.experimental.pallas import tpu as pltpu
from jax.experimental.pallas import tpu_sc as plsc

B, L, D = 16, 2048, 128
NC, NS, LANES = 2, 16, 16
NW = NC * NS
RW = (B * L) // NW
CH = 128
NCH = RW // CH
NSLOT = 4
WPB = L // 4
ALL_VALID = 0x01010101

_mesh = plsc.VectorSubcoreMesh(core_axis_name="c", subcore_axis_name="s")


@functools.partial(
    pl.kernel,
    out_type=jax.ShapeDtypeStruct((B * L, D), jnp.float32),
    mesh=_mesh,
    scratch_types=[
        pltpu.VMEM((WPB + LANES,), jnp.int32),
        pltpu.VMEM((NCH, CH), jnp.int32),
        pltpu.VMEM((NSLOT, CH, D), jnp.float32),
        pltpu.SemaphoreType.DMA,
        pltpu.SemaphoreType.DMA,
        pltpu.SemaphoreType.DMA,
        pltpu.SemaphoreType.DMA,
        pltpu.SemaphoreType.DMA,
        pltpu.SemaphoreType.DMA,
        pltpu.SemaphoreType.DMA,
        pltpu.SemaphoreType.DMA,
    ],
)
def _flatten_sc(maskw_hbm, ex_hbm, out_ex,
                maskw_v, idx_v, rows_v,
                gsem0, gsem1, gsem2, gsem3,
                wsem0, wsem1, wsem2, wsem3):
    gsem = (gsem0, gsem1, gsem2, gsem3)
    wsem = (wsem0, wsem1, wsem2, wsem3)
    wid = lax.axis_index("s") * NC + lax.axis_index("c")
    b = wid // 2
    half = wid % 2
    row0 = b * L + half * RW
    jbase = half * RW

    pltpu.sync_copy(maskw_hbm.at[b], maskw_v.at[pl.ds(0, WPB)])

    def _bs_body(_, carry):
        lo, hi = carry
        mid = (lo + hi) // 2
        go_right = maskw_v[pl.ds(mid, LANES)][0] == ALL_VALID
        return (jnp.where(go_right, mid + 1, lo),
                jnp.where(go_right, hi, mid))

    lo, _ = lax.fori_loop(0, 9, _bs_body, (jnp.int32(0), jnp.int32(WPB)))
    wsel = jnp.minimum(lo, WPB - 1)
    w = maskw_v[pl.ds(wsel, LANES)][0]
    extra = (w & 1) + ((w >> 8) & 1) + ((w >> 16) & 1) + ((w >> 24) & 1)
    nv = 4 * wsel + extra

    lane = lax.iota(jnp.int32, 16)

    def _build(c):
        def _row(i, _):
            off = pl.multiple_of(i * LANES, LANES)
            j = jbase + c * CH + off + lane
            idx_v[c, pl.ds(off, LANES)] = b * L + lax.rem(j, nv)
            return 0

        lax.fori_loop(0, CH // LANES, _row, 0)

    def _fire(c, s):
        is_identity = (jbase + c * CH + CH) <= nv

        @pl.when(is_identity)
        def _():
            pltpu.async_copy(
                ex_hbm.at[pl.ds(row0 + c * CH, CH)], rows_v.at[s], gsem[s])

        @pl.when(jnp.logical_not(is_identity))
        def _():
            pltpu.async_copy(ex_hbm.at[idx_v.at[c]], rows_v.at[s], gsem[s])

    def _wait_gather(s):
        pltpu.make_async_copy(
            ex_hbm.at[pl.ds(0, CH)], rows_v.at[s], gsem[s]).wait()

    writes = [None] * NCH
    for c in range(NSLOT):
        _build(c)
        _fire(c, c % NSLOT)
    for c in range(NSLOT, NCH):
        _build(c)
    for c in range(NCH):
        s = c % NSLOT
        _wait_gather(s)
        writes[c] = pltpu.async_copy(
            rows_v.at[s], out_ex.at[pl.ds(row0 + c * CH, CH)], wsem[s])
        if c + NSLOT < NCH:
            writes[c].wait()
            _fire(c + NSLOT, s)
    for c in range(NCH - NSLOT, NCH):
        writes[c].wait()


def _ctx_body(ctx_ref, out_ref):
    row = ctx_ref[pl.ds(pl.program_id(0), 1), :]
    out_ref[...] = jnp.broadcast_to(row, out_ref.shape)


_ctx_broadcast = pl.pallas_call(
    _ctx_body,
    grid=(B,),
    in_specs=[pl.BlockSpec((B, D), lambda i: (0, 0))],
    out_specs=pl.BlockSpec((L, D), lambda i: (i, 0)),
    out_shape=jax.ShapeDtypeStruct((B * L, D), jnp.float32),
)


def kernel(context_feature, example_feature, list_mask):
    maskw = list_mask.view(jnp.int32)
    ex2d = example_feature.reshape(B * L, D)
    flat_ex = _flatten_sc(maskw, ex2d)
    flat_ctx = _ctx_broadcast(context_feature)
    return flat_ctx, flat_ex


# speedup vs baseline: 1.2156x; 1.0358x over previous
import functools

import jax
import jax.numpy as jnp
from jax import lax
from jax.experimental import pallas as pl
from jax.experimental.pallas import tpu as pltpu
from jax.experimental.pallas import tpu_sc as plsc

B, L, D = 16, 2048, 128
NC, NS, LANES = 2, 16, 16
NW = NC * NS
RW = (B * L) // NW
CH = 128
NCH = RW // CH
NSLOT = 4
WPB = L // 4
ALL_VALID = 0x01010101

_mesh = plsc.VectorSubcoreMesh(core_axis_name="c", subcore_axis_name="s")


@functools.partial(
    pl.kernel,
    out_type=jax.ShapeDtypeStruct((B * L, D), jnp.float32),
    mesh=_mesh,
    scratch_types=[
        pltpu.VMEM((WPB + LANES,), jnp.int32),
        pltpu.VMEM((NCH, CH), jnp.int32),
        pltpu.VMEM((NSLOT, CH, D), jnp.float32),
        pltpu.SemaphoreType.DMA((NSLOT,)),
        pltpu.SemaphoreType.DMA((NSLOT,)),
    ],
)
def _flatten_sc(maskw_hbm, ex_hbm, out_ex,
                maskw_v, idx_v, rows_v, gsem, wsem):
    wid = lax.axis_index("s") * NC + lax.axis_index("c")
    b = wid // 2
    half = wid % 2
    row0 = b * L + half * RW
    jbase = half * RW

    pltpu.sync_copy(maskw_hbm.at[b], maskw_v.at[pl.ds(0, WPB)])

    def _bs_body(_, carry):
        lo, hi = carry
        mid = (lo + hi) // 2
        go_right = maskw_v[pl.ds(mid, LANES)][0] == ALL_VALID
        return (jnp.where(go_right, mid + 1, lo),
                jnp.where(go_right, hi, mid))

    lo, _ = lax.fori_loop(0, 9, _bs_body, (jnp.int32(0), jnp.int32(WPB)))
    wsel = jnp.minimum(lo, WPB - 1)
    w = maskw_v[pl.ds(wsel, LANES)][0]
    extra = (w & 1) + ((w >> 8) & 1) + ((w >> 16) & 1) + ((w >> 24) & 1)
    nv = 4 * wsel + extra

    lane = lax.iota(jnp.int32, 16)

    def _build(c):
        def _row(i, _):
            off = pl.multiple_of(i * LANES, LANES)
            j = jbase + c * CH + off + lane
            idx_v[c, pl.ds(off, LANES)] = b * L + lax.rem(j, nv)
            return 0

        lax.fori_loop(0, CH // LANES, _row, 0)

    def _fire(c, s):
        is_identity = (jbase + c * CH + CH) <= nv

        @pl.when(is_identity)
        def _():
            pltpu.async_copy(
                ex_hbm.at[pl.ds(row0 + c * CH, CH)], rows_v.at[s], gsem.at[s])

        @pl.when(jnp.logical_not(is_identity))
        def _():
            pltpu.async_copy(ex_hbm.at[idx_v.at[c]], rows_v.at[s],
                             gsem.at[s])

    def _wait_gather(s):
        pltpu.make_async_copy(
            ex_hbm.at[pl.ds(0, CH)], rows_v.at[s], gsem.at[s]).wait()

    def _wait_write(c, s):
        pltpu.make_async_copy(
            rows_v.at[s], out_ex.at[pl.ds(row0 + c * CH, CH)],
            wsem.at[s]).wait()

    def _prologue(c, _):
        _build(c)
        _fire(c, c % NSLOT)
        return 0

    lax.fori_loop(0, NSLOT, _prologue, 0)

    def _build_rest(c, _):
        _build(c)
        return 0

    lax.fori_loop(NSLOT, NCH, _build_rest, 0)

    def _steady(c, _):
        s = c % NSLOT
        _wait_gather(s)
        pltpu.async_copy(
            rows_v.at[s], out_ex.at[pl.ds(row0 + c * CH, CH)], wsem.at[s])

        @pl.when(c + NSLOT < NCH)
        def _():
            _wait_write(c, s)
            _fire(c + NSLOT, s)

        return 0

    lax.fori_loop(0, NCH, _steady, 0)

    def _drain(c, _):
        _wait_write(c, c % NSLOT)
        return 0

    lax.fori_loop(NCH - NSLOT, NCH, _drain, 0)


def _ctx_body(ctx_ref, out_ref):
    row = ctx_ref[pl.ds(pl.program_id(0), 1), :]
    out_ref[...] = jnp.broadcast_to(row, out_ref.shape)


_ctx_broadcast = pl.pallas_call(
    _ctx_body,
    grid=(B,),
    in_specs=[pl.BlockSpec((B, D), lambda i: (0, 0))],
    out_specs=pl.BlockSpec((L, D), lambda i: (i, 0)),
    out_shape=jax.ShapeDtypeStruct((B * L, D), jnp.float32),
)


def kernel(context_feature, example_feature, list_mask):
    maskw = list_mask.view(jnp.int32)
    ex2d = example_feature.reshape(B * L, D)
    flat_ex = _flatten_sc(maskw, ex2d)
    flat_ctx = _ctx_broadcast(context_feature)
    return flat_ctx, flat_ex
